# baseline (speedup 1.0000x reference)
import functools

import jax
import jax.numpy as jnp
import numpy as np
from jax import lax
from jax.experimental import pallas as pl
from jax.experimental.pallas import tpu as pltpu
from jax.experimental.pallas import tpu_sc as plsc

_B, _F, _S, _P, _E, _FF, _K = 64, 7, 512, 96, 8, 2048, 2
_N = _B * _F
_ALPHA = 10.0
_C = 4
_FC = _FF // _C
_NT = 4
_NG = _N // 16


def _logits_body(ti_hbm, gw_hbm, gb_hbm, out_ref, tif, gws, gbs,
                 sem_ti, sem_gw, sem_gb):
    cp_gw = pltpu.make_async_copy(gw_hbm, gws, sem_gw)
    cp_gb = pltpu.make_async_copy(gb_hbm, gbs, sem_gb)
    cp_gw.start()
    cp_gb.start()
    cps = [pltpu.make_async_copy(ti_hbm.at[i], tif.at[pl.ds(i * _B, _B), :],
                                 sem_ti) for i in range(_F)]
    for cp in cps:
        cp.start()
    for cp in cps:
        cp.wait()
    cp_gw.wait()
    cp_gb.wait()
    lt = lax.dot_general(gws[...], tif[...], (((1,), (1,)), ((), ())),
                         preferred_element_type=jnp.float32)
    lt = lt + gbs[...]
    for t in range(_NT - 1):
        out_ref[t] = lt[:, t * 128:(t + 1) * 128]
    out_ref[_NT - 1, :, 0:_N - 384] = lt[:, 384:_N]


def _sc_gating_body(logits_hbm, gates_hbm, lbuf, gbuf, sem):
    c = lax.axis_index("c")
    s = lax.axis_index("s")
    wid = s * 2 + c

    @pl.when(wid < _NT)
    def _():
        pltpu.async_copy(logits_hbm.at[wid], lbuf, sem).wait()
        for j in range(_E):
            sl = slice(j * 16, (j + 1) * 16)
            v = [lbuf[i, sl] for i in range(_E)]
            a = v[0]
            b = v[0] - jnp.float32(1e30)
            for i in range(1, _E):
                b = jnp.maximum(b, jnp.minimum(a, v[i]))
                a = jnp.maximum(a, v[i])
            ex = [jnp.exp(v[i] - a) for i in range(_E)]
            ssum = ex[0]
            for i in range(1, _E):
                ssum = ssum + ex[i]
            outv = []
            for i in range(_E):
                sm = ex[i] / ssum
                z = sm / (sm + 2.0)
                z2 = z * z
                ser = jnp.float32(1.0 / 15.0)
                for q in (13.0, 11.0, 9.0, 7.0, 5.0, 3.0, 1.0):
                    ser = ser * z2 + jnp.float32(1.0 / q)
                log1p = 2.0 * z * ser
                expm1 = jnp.exp(sm) - 1.0
                below = v[i] < b
                outv.append(jnp.where(below, _ALPHA * log1p, _ALPHA * expm1))
            mo = outv[0]
            for i in range(1, _E):
                mo = jnp.maximum(mo, outv[i])
            exo = [jnp.exp(outv[i] - mo) for i in range(_E)]
            s2 = exo[0]
            for i in range(1, _E):
                s2 = s2 + exo[i]
            for i in range(_E):
                gbuf[i, sl] = exo[i] / s2
        pltpu.async_copy(gbuf, gates_hbm.at[wid], sem).wait()


def _moe_body(x_hbm, g3_hbm, w1_ref, b1_hbm, w2_hbm, b2_hbm,
              out_ref, loss_ref,
              xf, b1s, b2s, w2s, gtr, acc_ref,
              sem_x, sem_g, sem_b1, sem_b2, sem_w2):
    e = pl.program_id(0)

    @pl.when(e == 0)
    def _setup():
        cp_w2 = pltpu.make_async_copy(w2_hbm, w2s, sem_w2)
        cp_b1 = pltpu.make_async_copy(b1_hbm, b1s, sem_b1)
        cp_b2 = pltpu.make_async_copy(b2_hbm, b2s, sem_b2)
        cp_g = pltpu.make_async_copy(g3_hbm, gtr, sem_g)
        cp_w2.start()
        cp_b1.start()
        cp_b2.start()
        cp_g.start()
        cps = [pltpu.make_async_copy(x_hbm.at[i],
                                     xf.at[pl.ds(i * _B, _B), :], sem_x)
               for i in range(_F)]
        for cp in cps:
            cp.start()
        cp_g.wait()

        g3v = gtr[...]
        gt = jnp.concatenate([g3v[t] for t in range(_NT)], axis=1)
        col = lax.broadcasted_iota(jnp.int32, (_F, _NT * 128), 1)
        rowi = lax.broadcasted_iota(jnp.int32, (_F, _NT * 128), 0)
        sel = jnp.where(col < _N, (col // _B == rowi).astype(jnp.float32), 0.0)
        gt_safe = jnp.where(
            lax.broadcasted_iota(jnp.int32, (_E, _NT * 128), 1) < _N, gt, 0.0)
        imp = lax.dot_general(gt_safe, sel, (((1,), (1,)), ((), ())),
                              preferred_element_type=jnp.float32)
        mean = jnp.mean(imp, axis=0, keepdims=True)
        var = jnp.sum((imp - mean) ** 2, axis=0, keepdims=True) / (_E - 1)
        loss_ref[...] = jnp.sum(var / (mean ** 2 + 1e-10),
                                keepdims=True).reshape(1, 1)

        for cp in cps:
            cp.wait()
        cp_b1.wait()
        cp_b2.wait()
        cp_w2.wait()

    erow = lax.broadcasted_iota(jnp.int32, (_E, 1), 0)
    b1row = jnp.sum(jnp.where(erow == e, b1s[...], 0.0), axis=0,
                    keepdims=True)

    g3v = gtr[...]
    grow_full = jnp.concatenate(
        [jnp.sum(jnp.where(erow == e, g3v[t], 0.0), axis=0, keepdims=True)
         for t in range(_NT)], axis=1)
    grow = grow_full[:, :_N]

    xb = xf[...].astype(jnp.bfloat16)
    o = jnp.zeros((_P, _N), jnp.float32)
    for c in range(_C):
        w1b = w1_ref[0, :, c * _FC:(c + 1) * _FC].astype(jnp.bfloat16)
        h = jnp.dot(xb, w1b, preferred_element_type=jnp.float32)
        h = h + b1row[:, c * _FC:(c + 1) * _FC]
        h = 0.5 * h * (1.0 + lax.erf(h * np.float32(1.0 / np.sqrt(2.0))))
        w2c = w2s[e, :, c * _FC:(c + 1) * _FC].astype(jnp.bfloat16)
        o = o + lax.dot_general(
            w2c, h.astype(jnp.bfloat16), (((1,), (1,)), ((), ())),
            preferred_element_type=jnp.float32)

    contrib = grow * o

    @pl.when(e == 0)
    def _init():
        gtall = jnp.concatenate([g3v[t] for t in range(_NT)],
                                axis=1)[:, :_N]
        bias_t = lax.dot_general(b2s[...], gtall, (((0,), (0,)), ((), ())),
                                 preferred_element_type=jnp.float32)
        acc_ref[...] = contrib + bias_t

    @pl.when(e > 0)
    def _acc():
        acc_ref[...] += contrib

    @pl.when(e == _E - 1)
    def _writeback():
        out_ref[...] = acc_ref[...]


def kernel(x, time_embedding, gate_W, gate_b, W1, b1, W2, b2):
    xt = jnp.transpose(x, (1, 0, 2))
    tit = jnp.transpose(time_embedding, (1, 0, 2))
    gwt = gate_W.T
    w2t = jnp.transpose(W2, (0, 2, 1))
    gb = gate_b.reshape(_E, 1)

    logits3 = pl.pallas_call(
        _logits_body,
        in_specs=[
            pl.BlockSpec(memory_space=pltpu.MemorySpace.HBM),
            pl.BlockSpec(memory_space=pltpu.MemorySpace.HBM),
            pl.BlockSpec(memory_space=pltpu.MemorySpace.HBM),
        ],
        out_specs=pl.BlockSpec((_NT, _E, 128), lambda: (0, 0, 0)),
        out_shape=jax.ShapeDtypeStruct((_NT, _E, 128), jnp.float32),
        scratch_shapes=[
            pltpu.VMEM((_N, _S), jnp.float32),
            pltpu.VMEM((_E, _S), jnp.float32),
            pltpu.VMEM((_E, 1), jnp.float32),
            pltpu.SemaphoreType.DMA,
            pltpu.SemaphoreType.DMA,
            pltpu.SemaphoreType.DMA,
        ],
        compiler_params=pltpu.CompilerParams(
            vmem_limit_bytes=61_000_000),
    )(tit, gwt, gb)

    sc_gating = functools.partial(
        pl.kernel,
        out_type=jax.ShapeDtypeStruct((_NT, _E, 128), jnp.float32),
        mesh=plsc.VectorSubcoreMesh(
            core_axis_name="c", subcore_axis_name="s"),
        scratch_types=[
            pltpu.VMEM((_E, 128), jnp.float32),
            pltpu.VMEM((_E, 128), jnp.float32),
            pltpu.SemaphoreType.DMA,
        ],
    )(_sc_gating_body)
    gates3 = sc_gating(logits3)

    out, loss = pl.pallas_call(
        _moe_body,
        grid=(_E,),
        in_specs=[
            pl.BlockSpec(memory_space=pltpu.MemorySpace.HBM),
            pl.BlockSpec(memory_space=pltpu.MemorySpace.HBM),
            pl.BlockSpec((1, _S, _FF), lambda e: (e, 0, 0)),
            pl.BlockSpec(memory_space=pltpu.MemorySpace.HBM),
            pl.BlockSpec(memory_space=pltpu.MemorySpace.HBM),
            pl.BlockSpec(memory_space=pltpu.MemorySpace.HBM),
        ],
        out_specs=[
            pl.BlockSpec((_P, _N), lambda e: (0, 0)),
            pl.BlockSpec((1, 1), lambda e: (0, 0)),
        ],
        out_shape=[
            jax.ShapeDtypeStruct((_P, _N), jnp.float32),
            jax.ShapeDtypeStruct((1, 1), jnp.float32),
        ],
        scratch_shapes=[
            pltpu.VMEM((_N, _S), jnp.float32),
            pltpu.VMEM((_E, _FF), jnp.float32),
            pltpu.VMEM((_E, _P), jnp.float32),
            pltpu.VMEM((_E, _P, _FF), jnp.float32),
            pltpu.VMEM((_NT, _E, 128), jnp.float32),
            pltpu.VMEM((_P, _N), jnp.float32),
            pltpu.SemaphoreType.DMA,
            pltpu.SemaphoreType.DMA,
            pltpu.SemaphoreType.DMA,
            pltpu.SemaphoreType.DMA,
            pltpu.SemaphoreType.DMA,
        ],
        compiler_params=pltpu.CompilerParams(
            dimension_semantics=("arbitrary",),
            vmem_limit_bytes=61_000_000),
    )(xt, gates3, W1, b1, w2t, b2)

    out = jnp.transpose(out.reshape(_P, _F, _B), (2, 1, 0))
    return out, loss[0, 0]


# speedup vs baseline: 1.6590x; 1.6590x over previous
import jax
import jax.numpy as jnp
import numpy as np
from jax import lax
from jax.experimental import pallas as pl
from jax.experimental.pallas import tpu as pltpu

_B, _F, _S, _P, _E, _FF, _K = 64, 7, 512, 96, 8, 2048, 2
_N = _B * _F
_ALPHA = 10.0
_C = 4
_FC = _FF // _C


def _moe_body(x_hbm, ti_hbm, gw_hbm, gb_hbm, w1_ref, b1_hbm, w2_hbm, b2_hbm,
              out_ref, loss_ref,
              xf, tif, gws, gbs, b1s, b2s, w2s, gates_ref, acc_ref,
              sem_x, sem_ti, sem_gw, sem_gb, sem_b1, sem_b2, sem_w2):
    s_id = pl.program_id(0)
    e = s_id // 2
    half = s_id % 2

    @pl.when(s_id == 0)
    def _setup_and_gating():
        cp_w2 = pltpu.make_async_copy(w2_hbm, w2s, sem_w2)
        cp_gw = pltpu.make_async_copy(gw_hbm, gws, sem_gw)
        cp_gb = pltpu.make_async_copy(gb_hbm, gbs, sem_gb)
        cp_b1 = pltpu.make_async_copy(b1_hbm, b1s, sem_b1)
        cp_b2 = pltpu.make_async_copy(b2_hbm, b2s, sem_b2)
        cp_w2.start()
        cp_gw.start()
        cp_gb.start()
        cp_b1.start()
        cp_b2.start()
        cps_x = [pltpu.make_async_copy(
            x_hbm.at[i], xf.at[pl.ds(i * _B, _B), :], sem_x)
            for i in range(_F)]
        cps_ti = [pltpu.make_async_copy(
            ti_hbm.at[i], tif.at[pl.ds(i * _B, _B), :], sem_ti)
            for i in range(_F)]
        for cp in cps_x + cps_ti:
            cp.start()
        for cp in cps_x + cps_ti:
            cp.wait()

        cp_gw.wait()
        cp_gb.wait()
        logits = lax.dot_general(
            tif[...], gws[...], (((1,), (1,)), ((), ())),
            preferred_element_type=jnp.float32) + gbs[...]
        m1 = jnp.max(logits, axis=1, keepdims=True)
        idx = lax.broadcasted_iota(jnp.int32, (_N, _E), 1)
        first_idx = jnp.min(jnp.where(logits == m1, idx, _E), axis=1,
                            keepdims=True)
        m2 = jnp.max(jnp.where(idx == first_idx, -jnp.inf, logits), axis=1,
                     keepdims=True)
        below_topk = logits < m2
        ex = jnp.exp(logits - m1)
        sm = ex / jnp.sum(ex, axis=1, keepdims=True)
        outv = jnp.where(below_topk, _ALPHA * jnp.log(sm + 1.0),
                         _ALPHA * (jnp.exp(sm) - 1.0))
        mo = jnp.max(outv, axis=1, keepdims=True)
        exo = jnp.exp(outv - mo)
        gates = exo / jnp.sum(exo, axis=1, keepdims=True)
        gates_ref[...] = gates

        row = lax.broadcasted_iota(jnp.int32, (_F, _N), 0)
        col = lax.broadcasted_iota(jnp.int32, (_F, _N), 1)
        sel = (col // _B == row).astype(jnp.float32)
        imp = jnp.dot(sel, gates, preferred_element_type=jnp.float32)
        mean = jnp.mean(imp, axis=1, keepdims=True)
        var = jnp.sum((imp - mean) ** 2, axis=1, keepdims=True) / (_E - 1)
        loss_ref[...] = jnp.sum(var / (mean ** 2 + 1e-10),
                                keepdims=True).reshape(1, 1)

        cp_b1.wait()
        cp_b2.wait()
        cp_w2.wait()

    erow = lax.broadcasted_iota(jnp.int32, (_E, 1), 0)
    b1row = jnp.sum(jnp.where(erow == e, b1s[...], 0.0), axis=0,
                    keepdims=True)
    b2row = jnp.sum(jnp.where(erow == e, b2s[...], 0.0), axis=0,
                    keepdims=True)

    xb = xf[...].astype(jnp.bfloat16)
    halff = half.astype(jnp.float32)
    o = b2row * halff
    for c in range(_C // 2):
        lo = c * _FC
        hi_ = (_FF // 2) + c * _FC
        w1b = w1_ref[0, :, lo:lo + _FC].astype(jnp.bfloat16)
        h = jnp.dot(xb, w1b, preferred_element_type=jnp.float32)
        b1c = (b1row[:, lo:lo + _FC] * (1.0 - halff)
               + b1row[:, hi_:hi_ + _FC] * halff)
        h = h + b1c
        h = 0.5 * h * (1.0 + lax.erf(h * np.float32(1.0 / np.sqrt(2.0))))
        w2c = (w2s[e, :, lo:lo + _FC] * (1.0 - halff)
               + w2s[e, :, hi_:hi_ + _FC] * halff).astype(jnp.bfloat16)
        o = o + lax.dot_general(
            h.astype(jnp.bfloat16), w2c, (((1,), (1,)), ((), ())),
            preferred_element_type=jnp.float32)

    lane = lax.broadcasted_iota(jnp.int32, (_N, _E), 1)
    g = jnp.sum(jnp.where(lane == e, gates_ref[...], 0.0), axis=1,
                keepdims=True)
    contrib = g * o

    @pl.when(s_id == 0)
    def _init():
        acc_ref[...] = contrib

    @pl.when(s_id > 0)
    def _acc():
        acc_ref[...] += contrib

    @pl.when(s_id == 2 * _E - 1)
    def _writeback():
        for i in range(_F):
            out_ref[i] = acc_ref[pl.ds(i * _B, _B), :]


def kernel(x, time_embedding, gate_W, gate_b, W1, b1, W2, b2):
    xt = jnp.transpose(x, (1, 0, 2))
    tit = jnp.transpose(time_embedding, (1, 0, 2))
    gwt = gate_W.T
    w2t = jnp.transpose(W2, (0, 2, 1))
    gb = gate_b.reshape(1, _E)

    out, loss = pl.pallas_call(
        _moe_body,
        grid=(2 * _E,),
        in_specs=[
            pl.BlockSpec(memory_space=pltpu.MemorySpace.HBM),
            pl.BlockSpec(memory_space=pltpu.MemorySpace.HBM),
            pl.BlockSpec(memory_space=pltpu.MemorySpace.HBM),
            pl.BlockSpec(memory_space=pltpu.MemorySpace.HBM),
            pl.BlockSpec((1, _S, _FF // 2),
                         lambda s: (s // 2, 0, s % 2)),
            pl.BlockSpec(memory_space=pltpu.MemorySpace.HBM),
            pl.BlockSpec(memory_space=pltpu.MemorySpace.HBM),
            pl.BlockSpec(memory_space=pltpu.MemorySpace.HBM),
        ],
        out_specs=[
            pl.BlockSpec((_F, _B, _P), lambda e: (0, 0, 0)),
            pl.BlockSpec((1, 1), lambda e: (0, 0)),
        ],
        out_shape=[
            jax.ShapeDtypeStruct((_F, _B, _P), jnp.float32),
            jax.ShapeDtypeStruct((1, 1), jnp.float32),
        ],
        scratch_shapes=[
            pltpu.VMEM((_N, _S), jnp.float32),
            pltpu.VMEM((_N, _S), jnp.float32),
            pltpu.VMEM((_E, _S), jnp.float32),
            pltpu.VMEM((1, _E), jnp.float32),
            pltpu.VMEM((_E, _FF), jnp.float32),
            pltpu.VMEM((_E, _P), jnp.float32),
            pltpu.VMEM((_E, _P, _FF), jnp.float32),
            pltpu.VMEM((_N, _E), jnp.float32),
            pltpu.VMEM((_N, _P), jnp.float32),
            pltpu.SemaphoreType.DMA,
            pltpu.SemaphoreType.DMA,
            pltpu.SemaphoreType.DMA,
            pltpu.SemaphoreType.DMA,
            pltpu.SemaphoreType.DMA,
            pltpu.SemaphoreType.DMA,
            pltpu.SemaphoreType.DMA,
        ],
        compiler_params=pltpu.CompilerParams(
            dimension_semantics=("arbitrary",),
            vmem_limit_bytes=61_000_000),
    )(xt, tit, gwt, gb, W1, b1, w2t, b2)

    return jnp.transpose(out, (1, 0, 2)), loss[0, 0]


# speedup vs baseline: 2.0185x; 1.2167x over previous
import jax
import jax.numpy as jnp
import numpy as np
from jax import lax
from jax.experimental import pallas as pl
from jax.experimental.pallas import tpu as pltpu

_B, _F, _S, _P, _E, _FF, _K = 64, 7, 512, 96, 8, 2048, 2
_N = _B * _F
_ALPHA = 10.0
_C = 4
_FC = _FF // _C


def _moe_body(x_hbm, ti_hbm, gw_hbm, gb_hbm, w1_ref, b1_hbm, w2_hbm, b2_hbm,
              out_ref, loss_ref,
              xf, tif, gws, gbs, b1s, b2s, w2s, gates_ref, acc_ref,
              sem_x, sem_ti, sem_gw, sem_gb, sem_b1, sem_b2, sem_w2):
    e = pl.program_id(0)

    @pl.when(e == 0)
    def _setup_and_gating():
        cp_w2 = pltpu.make_async_copy(w2_hbm, w2s, sem_w2)
        cp_gw = pltpu.make_async_copy(gw_hbm, gws, sem_gw)
        cp_gb = pltpu.make_async_copy(gb_hbm, gbs, sem_gb)
        cp_b1 = pltpu.make_async_copy(b1_hbm, b1s, sem_b1)
        cp_b2 = pltpu.make_async_copy(b2_hbm, b2s, sem_b2)
        cp_w2.start()
        cp_gw.start()
        cp_gb.start()
        cp_b1.start()
        cp_b2.start()
        cps_x = [pltpu.make_async_copy(
            x_hbm.at[i], xf.at[pl.ds(i * _B, _B), :], sem_x)
            for i in range(_F)]
        cps_ti = [pltpu.make_async_copy(
            ti_hbm.at[i], tif.at[pl.ds(i * _B, _B), :], sem_ti)
            for i in range(_F)]
        for cp in cps_x + cps_ti:
            cp.start()
        for cp in cps_x + cps_ti:
            cp.wait()

        cp_gw.wait()
        cp_gb.wait()
        logits = lax.dot_general(
            tif[...], gws[...], (((1,), (1,)), ((), ())),
            preferred_element_type=jnp.float32) + gbs[...]
        m1 = jnp.max(logits, axis=1, keepdims=True)
        idx = lax.broadcasted_iota(jnp.int32, (_N, _E), 1)
        first_idx = jnp.min(jnp.where(logits == m1, idx, _E), axis=1,
                            keepdims=True)
        m2 = jnp.max(jnp.where(idx == first_idx, -jnp.inf, logits), axis=1,
                     keepdims=True)
        below_topk = logits < m2
        ex = jnp.exp(logits - m1)
        sm = ex / jnp.sum(ex, axis=1, keepdims=True)
        outv = jnp.where(below_topk, _ALPHA * jnp.log(sm + 1.0),
                         _ALPHA * (jnp.exp(sm) - 1.0))
        mo = jnp.max(outv, axis=1, keepdims=True)
        exo = jnp.exp(outv - mo)
        gates = exo / jnp.sum(exo, axis=1, keepdims=True)
        gates_ref[...] = gates

        row = lax.broadcasted_iota(jnp.int32, (_F, _N), 0)
        col = lax.broadcasted_iota(jnp.int32, (_F, _N), 1)
        sel = (col // _B == row).astype(jnp.float32)
        imp = jnp.dot(sel, gates, preferred_element_type=jnp.float32)
        mean = jnp.mean(imp, axis=1, keepdims=True)
        var = jnp.sum((imp - mean) ** 2, axis=1, keepdims=True) / (_E - 1)
        loss_ref[...] = jnp.sum(var / (mean ** 2 + 1e-10),
                                keepdims=True).reshape(1, 1)

        cp_b1.wait()
        cp_b2.wait()
        cp_w2.wait()

    erow = lax.broadcasted_iota(jnp.int32, (_E, 1), 0)
    b1row = jnp.sum(jnp.where(erow == e, b1s[...], 0.0), axis=0,
                    keepdims=True)
    b2row = jnp.sum(jnp.where(erow == e, b2s[...], 0.0), axis=0,
                    keepdims=True)

    xb = xf[...].astype(jnp.bfloat16)
    o = b2row
    for c in range(_C):
        w1b = w1_ref[0, :, c * _FC:(c + 1) * _FC].astype(jnp.bfloat16)
        h = jnp.dot(xb, w1b, preferred_element_type=jnp.float32)
        h = h + b1row[:, c * _FC:(c + 1) * _FC]
        h = 0.5 * h * (1.0 + lax.erf(h * np.float32(1.0 / np.sqrt(2.0))))
        w2c = w2s[e, :, c * _FC:(c + 1) * _FC].astype(jnp.bfloat16)
        o = o + lax.dot_general(
            h.astype(jnp.bfloat16), w2c, (((1,), (1,)), ((), ())),
            preferred_element_type=jnp.float32)

    lane = lax.broadcasted_iota(jnp.int32, (_N, _E), 1)
    g = jnp.sum(jnp.where(lane == e, gates_ref[...], 0.0), axis=1,
                keepdims=True)
    contrib = g * o

    @pl.when(e == 0)
    def _init():
        acc_ref[...] = contrib

    @pl.when(e > 0)
    def _acc():
        acc_ref[...] += contrib

    @pl.when(e == _E - 1)
    def _writeback():
        for i in range(_F):
            out_ref[i] = acc_ref[pl.ds(i * _B, _B), :]


def kernel(x, time_embedding, gate_W, gate_b, W1, b1, W2, b2):
    xt = jnp.transpose(x, (1, 0, 2))
    tit = jnp.transpose(time_embedding, (1, 0, 2))
    gwt = gate_W.T
    w2t = jnp.transpose(W2, (0, 2, 1))
    gb = gate_b.reshape(1, _E)

    out, loss = pl.pallas_call(
        _moe_body,
        grid=(_E,),
        in_specs=[
            pl.BlockSpec(memory_space=pltpu.MemorySpace.HBM),
            pl.BlockSpec(memory_space=pltpu.MemorySpace.HBM),
            pl.BlockSpec(memory_space=pltpu.MemorySpace.HBM),
            pl.BlockSpec(memory_space=pltpu.MemorySpace.HBM),
            pl.BlockSpec((1, _S, _FF), lambda e: (e, 0, 0)),
            pl.BlockSpec(memory_space=pltpu.MemorySpace.HBM),
            pl.BlockSpec(memory_space=pltpu.MemorySpace.HBM),
            pl.BlockSpec(memory_space=pltpu.MemorySpace.HBM),
        ],
        out_specs=[
            pl.BlockSpec((_F, _B, _P), lambda e: (0, 0, 0)),
            pl.BlockSpec((1, 1), lambda e: (0, 0)),
        ],
        out_shape=[
            jax.ShapeDtypeStruct((_F, _B, _P), jnp.float32),
            jax.ShapeDtypeStruct((1, 1), jnp.float32),
        ],
        scratch_shapes=[
            pltpu.VMEM((_N, _S), jnp.float32),
            pltpu.VMEM((_N, _S), jnp.float32),
            pltpu.VMEM((_E, _S), jnp.float32),
            pltpu.VMEM((1, _E), jnp.float32),
            pltpu.VMEM((_E, _FF), jnp.float32),
            pltpu.VMEM((_E, _P), jnp.float32),
            pltpu.VMEM((_E, _P, _FF), jnp.float32),
            pltpu.VMEM((_N, _E), jnp.float32),
            pltpu.VMEM((_N, _P), jnp.float32),
            pltpu.SemaphoreType.DMA,
            pltpu.SemaphoreType.DMA,
            pltpu.SemaphoreType.DMA,
            pltpu.SemaphoreType.DMA,
            pltpu.SemaphoreType.DMA,
            pltpu.SemaphoreType.DMA,
            pltpu.SemaphoreType.DMA,
        ],
        compiler_params=pltpu.CompilerParams(
            dimension_semantics=("arbitrary",),
            vmem_limit_bytes=61_000_000),
    )(xt, tit, gwt, gb, W1, b1, w2t, b2)

    return jnp.transpose(out, (1, 0, 2)), loss[0, 0]
